# Initial kernel scaffold; baseline (speedup 1.0000x reference)
import jax
import jax.numpy as jnp
from jax.experimental import pallas as pl


def kernel(x, y, ind_x, px_r, W):
    raise NotImplementedError("write your pallas kernel here")


# speedup vs baseline: 2.2721x; 2.2721x over previous
import functools

import jax
import jax.numpy as jnp
from jax.experimental import pallas as pl
from jax.experimental.pallas import tpu as pltpu

N_INPUT = 20000
N_LABELS = 64
BATCH = 4096
EPS = 1e-8
B_TILE = 32

_LANCZOS = (
    676.5203681218851,
    -1259.1392167224028,
    771.3234287776531,
    -176.6150291621406,
    12.507343278686905,
    -0.13857109526572012,
    9.984369578019572e-6,
    1.5056327351493116e-7,
)
_LOG_SQRT_2PI = 0.9189385332046727
_LOG_7P5 = 2.0149030205422647


def _lanczos_lgamma1p(z):
    s = jnp.float32(0.9999999999998099)
    for k, c in enumerate(_LANCZOS):
        s = s + c / (z + jnp.float32(k + 1))
    t = z + 7.5
    log_t = _LOG_7P5 + jnp.log1p(z / 7.5)
    return _LOG_SQRT_2PI + (z + 0.5) * log_t - t + jnp.log(s)


def _lgamma(a):
    shift = a < 0.5
    z = jnp.where(shift, a, a - 1.0)
    log_y = _lanczos_lgamma1p(z)
    return jnp.where(shift, log_y - jnp.log(a), log_y)


def _lgamma_ge_half(a):
    return _lanczos_lgamma1p(a - 1.0)


def _loss_kernel(x_ref, y_ref, pxr_ref, wt_ref, out_ref, sp_ref, th_ref, c_ref):
    i = pl.program_id(0)

    @pl.when(i == 0)
    def _init():
        sp_ref[...] = jax.nn.softplus(wt_ref[...])
        theta = jnp.exp(pxr_ref[...])
        th_ref[...] = theta
        c_ref[0, 0] = jnp.sum(
            theta * jnp.log(theta + EPS) - _lgamma(theta)
        )

    xb = x_ref[...]
    yb = y_ref[0, 0, :]
    labels = jax.lax.broadcasted_iota(jnp.int32, (B_TILE, N_LABELS), 1)
    onehot = (yb[:, None] == labels).astype(jnp.float32)
    px_scale = jax.lax.dot_general(
        onehot,
        sp_ref[...],
        dimension_numbers=(((1,), (0,)), ((), ())),
        preferred_element_type=jnp.float32,
        precision=jax.lax.Precision.HIGHEST,
    )

    lib = jnp.sum(xb, axis=1, keepdims=True)
    mu = lib * px_scale
    theta = th_ref[...]
    ltm = jnp.log(theta + mu + EPS)
    res = (
        -(theta + xb) * ltm
        + xb * jnp.log(mu + EPS)
        + _lgamma(xb + theta)
        - _lgamma_ge_half(xb + 1.0)
    )
    out_ref[0, 0, :] = -(jnp.sum(res, axis=1) + c_ref[0, 0])


@functools.partial(jax.jit, static_argnames=("interpret",))
def _run(x, y, px_r, W, interpret=False):
    nb = BATCH // B_TILE
    y2 = y.reshape(nb, 1, B_TILE)
    pxr2 = px_r.reshape(1, N_INPUT)
    wt = W.T

    loss = pl.pallas_call(
        _loss_kernel,
        grid=(nb,),
        in_specs=[
            pl.BlockSpec((B_TILE, N_INPUT), lambda i: (i, 0)),
            pl.BlockSpec((1, 1, B_TILE), lambda i: (i, 0, 0)),
            pl.BlockSpec((1, N_INPUT), lambda i: (0, 0)),
            pl.BlockSpec((N_LABELS, N_INPUT), lambda i: (0, 0)),
        ],
        out_specs=pl.BlockSpec((1, 1, B_TILE), lambda i: (i, 0, 0)),
        out_shape=jax.ShapeDtypeStruct((nb, 1, B_TILE), jnp.float32),
        scratch_shapes=[
            pltpu.VMEM((N_LABELS, N_INPUT), jnp.float32),
            pltpu.VMEM((1, N_INPUT), jnp.float32),
            pltpu.SMEM((1, 1), jnp.float32),
        ],
        interpret=interpret,
    )(x, y2, pxr2, wt)
    return loss.reshape(BATCH)


def kernel(x, y, ind_x, px_r, W):
    loss = _run(x, y, px_r, W)
    zero = jnp.asarray(0.0, dtype=jnp.float32)
    return (loss, zero, zero)


# speedup vs baseline: 3.2785x; 1.4429x over previous
import functools

import jax
import jax.numpy as jnp
from jax.experimental import pallas as pl
from jax.experimental.pallas import tpu as pltpu

N_INPUT = 20000
N_LABELS = 64
BATCH = 4096
EPS = 1e-8
B_TILE = 32

_LG_N = (
    75122.6331530452,
    80916.62789524844,
    36308.29514770108,
    8687.245297053594,
    1168.9264947922106,
    83.86760434239518,
    2.5066282751072975,
)
_LG_D = (720.0, 1764.0, 1624.0, 735.0, 175.0, 21.0, 1.0)


def _lgamma_pos(a):
    n = jnp.float32(_LG_N[-1])
    for c in _LG_N[-2::-1]:
        n = n * a + jnp.float32(c)
    d = jnp.float32(_LG_D[-1])
    for c in _LG_D[-2::-1]:
        d = d * a + jnp.float32(c)
    d = d * a
    t = a + 5.5
    return (a + 0.5) * jnp.log(t) - t + jnp.log(n / d)


def _loss_kernel(
    x_ref, y_ref, pxr_ref, wt_ref, out_ref, hi_ref, md_ref, lo_ref, th_ref, c_ref
):
    i = pl.program_id(0)

    @pl.when(i == 0)
    def _init():
        sp = jax.nn.softplus(wt_ref[...])
        hi = sp.astype(jnp.bfloat16)
        r1 = sp - hi.astype(jnp.float32)
        md = r1.astype(jnp.bfloat16)
        lo = (r1 - md.astype(jnp.float32)).astype(jnp.bfloat16)
        hi_ref[...] = hi
        md_ref[...] = md
        lo_ref[...] = lo
        theta = jnp.exp(pxr_ref[...])
        th_ref[...] = theta
        c_ref[0, 0] = jnp.sum(theta * jnp.log(theta + EPS) - _lgamma_pos(theta))

    xb = x_ref[...]
    yb = y_ref[0, 0, :]
    labels = jax.lax.broadcasted_iota(jnp.int32, (B_TILE, N_LABELS), 1)
    onehot = (yb[:, None] == labels).astype(jnp.bfloat16)

    def _sel(ref):
        return jax.lax.dot_general(
            onehot,
            ref[...],
            dimension_numbers=(((1,), (0,)), ((), ())),
            preferred_element_type=jnp.float32,
        )

    px_scale = (_sel(hi_ref) + _sel(md_ref)) + _sel(lo_ref)

    lib = jnp.sum(xb, axis=1, keepdims=True)
    mu = lib * px_scale
    theta = th_ref[...]
    ltm = jnp.log(theta + mu + EPS)
    res = (
        -(theta + xb) * ltm
        + xb * jnp.log(mu + EPS)
        + _lgamma_pos(xb + theta)
        - _lgamma_pos(xb + 1.0)
    )
    out_ref[0, 0, :] = -(jnp.sum(res, axis=1) + c_ref[0, 0])


@functools.partial(jax.jit, static_argnames=("interpret",))
def _run(x, y, px_r, W, interpret=False):
    nb = BATCH // B_TILE
    y2 = y.reshape(nb, 1, B_TILE)
    pxr2 = px_r.reshape(1, N_INPUT)
    wt = W.T

    loss = pl.pallas_call(
        _loss_kernel,
        grid=(nb,),
        in_specs=[
            pl.BlockSpec((B_TILE, N_INPUT), lambda i: (i, 0)),
            pl.BlockSpec((1, 1, B_TILE), lambda i: (i, 0, 0)),
            pl.BlockSpec((1, N_INPUT), lambda i: (0, 0)),
            pl.BlockSpec((N_LABELS, N_INPUT), lambda i: (0, 0)),
        ],
        out_specs=pl.BlockSpec((1, 1, B_TILE), lambda i: (i, 0, 0)),
        out_shape=jax.ShapeDtypeStruct((nb, 1, B_TILE), jnp.float32),
        scratch_shapes=[
            pltpu.VMEM((N_LABELS, N_INPUT), jnp.bfloat16),
            pltpu.VMEM((N_LABELS, N_INPUT), jnp.bfloat16),
            pltpu.VMEM((N_LABELS, N_INPUT), jnp.bfloat16),
            pltpu.VMEM((1, N_INPUT), jnp.float32),
            pltpu.SMEM((1, 1), jnp.float32),
        ],
        interpret=interpret,
    )(x, y2, pxr2, wt)
    return loss.reshape(BATCH)


def kernel(x, y, ind_x, px_r, W):
    loss = _run(x, y, px_r, W)
    zero = jnp.asarray(0.0, dtype=jnp.float32)
    return (loss, zero, zero)


# speedup vs baseline: 3.8830x; 1.1844x over previous
import functools

import jax
import jax.numpy as jnp
from jax.experimental import pallas as pl
from jax.experimental.pallas import tpu as pltpu

N_INPUT = 20000
N_LABELS = 64
BATCH = 4096
EPS = 1e-8
B_TILE = 32

_LG_N = (
    655.1778003977308,
    651.7861284548891,
    243.1516405664637,
    40.31491809436625,
    2.5066282746310007,
)
_LG_D = (6.0, 11.0, 6.0, 1.0)


def _lgamma_pos(a):
    n = jnp.float32(_LG_N[-1])
    for c in _LG_N[-2::-1]:
        n = n * a + jnp.float32(c)
    d = jnp.float32(_LG_D[-1])
    for c in _LG_D[-2::-1]:
        d = d * a + jnp.float32(c)
    d = d * a
    t = a + 4.0
    return (a - 0.5) * jnp.log(t) - t + jnp.log(n / d)


def _loss_kernel(
    x_ref, y_ref, pxr_ref, wt_ref, out_ref, hi_ref, md_ref, lo_ref, th_ref, c_ref
):
    i = pl.program_id(0)

    @pl.when(i == 0)
    def _init():
        sp = jax.nn.softplus(wt_ref[...])
        hi = sp.astype(jnp.bfloat16)
        r1 = sp - hi.astype(jnp.float32)
        md = r1.astype(jnp.bfloat16)
        lo = (r1 - md.astype(jnp.float32)).astype(jnp.bfloat16)
        hi_ref[...] = hi
        md_ref[...] = md
        lo_ref[...] = lo
        theta = jnp.exp(pxr_ref[...])
        th_ref[...] = theta
        c_ref[0, 0] = jnp.sum(theta * jnp.log(theta + EPS) - _lgamma_pos(theta))

    xb = x_ref[...]
    yb = y_ref[0, 0, :]
    labels = jax.lax.broadcasted_iota(jnp.int32, (B_TILE, N_LABELS), 1)
    onehot = (yb[:, None] == labels).astype(jnp.bfloat16)

    def _sel(ref):
        return jax.lax.dot_general(
            onehot,
            ref[...],
            dimension_numbers=(((1,), (0,)), ((), ())),
            preferred_element_type=jnp.float32,
        )

    px_scale = (_sel(hi_ref) + _sel(md_ref)) + _sel(lo_ref)

    lib = jnp.sum(xb, axis=1, keepdims=True)
    mu = lib * px_scale
    theta = th_ref[...]
    ltm = jnp.log(theta + mu + EPS)
    res = (
        -(theta + xb) * ltm
        + xb * jnp.log(mu + EPS)
        + _lgamma_pos(xb + theta)
        - _lgamma_pos(xb + 1.0)
    )
    out_ref[0, 0, :] = -(jnp.sum(res, axis=1) + c_ref[0, 0])


@functools.partial(jax.jit, static_argnames=("interpret",))
def _run(x, y, px_r, W, interpret=False):
    nb = BATCH // B_TILE
    y2 = y.reshape(nb, 1, B_TILE)
    pxr2 = px_r.reshape(1, N_INPUT)
    wt = W.T

    loss = pl.pallas_call(
        _loss_kernel,
        grid=(nb,),
        in_specs=[
            pl.BlockSpec((B_TILE, N_INPUT), lambda i: (i, 0)),
            pl.BlockSpec((1, 1, B_TILE), lambda i: (i, 0, 0)),
            pl.BlockSpec((1, N_INPUT), lambda i: (0, 0)),
            pl.BlockSpec((N_LABELS, N_INPUT), lambda i: (0, 0)),
        ],
        out_specs=pl.BlockSpec((1, 1, B_TILE), lambda i: (i, 0, 0)),
        out_shape=jax.ShapeDtypeStruct((nb, 1, B_TILE), jnp.float32),
        scratch_shapes=[
            pltpu.VMEM((N_LABELS, N_INPUT), jnp.bfloat16),
            pltpu.VMEM((N_LABELS, N_INPUT), jnp.bfloat16),
            pltpu.VMEM((N_LABELS, N_INPUT), jnp.bfloat16),
            pltpu.VMEM((1, N_INPUT), jnp.float32),
            pltpu.SMEM((1, 1), jnp.float32),
        ],
        interpret=interpret,
    )(x, y2, pxr2, wt)
    return loss.reshape(BATCH)


def kernel(x, y, ind_x, px_r, W):
    loss = _run(x, y, px_r, W)
    zero = jnp.asarray(0.0, dtype=jnp.float32)
    return (loss, zero, zero)


# speedup vs baseline: 4.7275x; 1.2175x over previous
import functools

import jax
import jax.numpy as jnp
from jax.experimental import pallas as pl
from jax.experimental.pallas import tpu as pltpu

N_INPUT = 20000
N_LABELS = 64
BATCH = 4096
EPS = 1e-8
B_TILE = 32

_LG_N = (
    655.1778003977308,
    651.7861284548891,
    243.1516405664637,
    40.31491809436625,
    2.5066282746310007,
)
_LG_D = (6.0, 11.0, 6.0, 1.0)


def _lgamma_pos(a):
    n = jnp.float32(_LG_N[-1])
    for c in _LG_N[-2::-1]:
        n = n * a + jnp.float32(c)
    d = jnp.float32(_LG_D[-1])
    for c in _LG_D[-2::-1]:
        d = d * a + jnp.float32(c)
    d = d * a
    t = a + 4.0
    return (a - 0.5) * jnp.log(t) - t + jnp.log(n / d)


_SP3_P = (10.449703348243359, 10.238049794415314, 2.5066282746310007)
_SP3_PS = (23.19438141728967, 15.251306343677316, 2.5066282746310007)


def _loss_kernel(
    x_ref, y_ref, pxr_ref, wt_ref, out_ref, hi_ref, md_ref, lo_ref, th_ref, c_ref
):
    i = pl.program_id(0)

    @pl.when(i == 0)
    def _init():
        sp = jax.nn.softplus(wt_ref[...])
        hi = sp.astype(jnp.bfloat16)
        r1 = sp - hi.astype(jnp.float32)
        md = r1.astype(jnp.bfloat16)
        lo = (r1 - md.astype(jnp.float32)).astype(jnp.bfloat16)
        hi_ref[...] = hi
        md_ref[...] = md
        lo_ref[...] = lo
        theta = jnp.exp(pxr_ref[...])
        th_ref[...] = theta
        c_ref[0, 0] = jnp.sum(
            theta * jnp.log(theta + EPS) - _lgamma_pos(theta) - theta + 1.0
        )

    xb = x_ref[...]
    yb = y_ref[0, 0, :]
    labels = jax.lax.broadcasted_iota(jnp.int32, (B_TILE, N_LABELS), 1)
    onehot = (yb[:, None] == labels).astype(jnp.bfloat16)

    def _sel(ref):
        return jax.lax.dot_general(
            onehot,
            ref[...],
            dimension_numbers=(((1,), (0,)), ((), ())),
            preferred_element_type=jnp.float32,
        )

    px_scale = (_sel(hi_ref) + _sel(md_ref)) + _sel(lo_ref)

    th = th_ref[...]
    te = th + EPS
    th2 = th + 2.0

    lib = jnp.sum(xb, axis=1, keepdims=True)
    mu = lib * px_scale
    a1 = xb + th
    t1 = xb + th2
    t2 = xb + 3.0
    p1 = (_SP3_P[2] * a1 + _SP3_P[1]) * a1 + _SP3_P[0]
    q1 = a1 * (a1 + 1.0)
    p2 = (_SP3_PS[2] * xb + _SP3_PS[1]) * xb + _SP3_PS[0]
    v = xb + 1.5
    q2 = v * v - 0.25
    ltm = jnp.log(te + mu)
    lmu = jnp.log(mu + EPS)
    lt1 = jnp.log(t1)
    lt2 = jnp.log(t2)
    lr1 = jnp.log(p1 / q1)
    lr2 = jnp.log(p2 / q2)
    contrib = (
        a1 * (lt1 - ltm)
        + xb * (lmu - lt2)
        - 0.5 * (lt1 + lt2)
        + (lr1 - lr2)
    )
    out_ref[0, 0, :] = -(jnp.sum(contrib, axis=1) + c_ref[0, 0])


@functools.partial(jax.jit, static_argnames=("interpret",))
def _run(x, y, px_r, W, interpret=False):
    nb = BATCH // B_TILE
    y2 = y.reshape(nb, 1, B_TILE)
    pxr2 = px_r.reshape(1, N_INPUT)
    wt = W.T

    loss = pl.pallas_call(
        _loss_kernel,
        grid=(nb,),
        in_specs=[
            pl.BlockSpec((B_TILE, N_INPUT), lambda i: (i, 0)),
            pl.BlockSpec((1, 1, B_TILE), lambda i: (i, 0, 0)),
            pl.BlockSpec((1, N_INPUT), lambda i: (0, 0)),
            pl.BlockSpec((N_LABELS, N_INPUT), lambda i: (0, 0)),
        ],
        out_specs=pl.BlockSpec((1, 1, B_TILE), lambda i: (i, 0, 0)),
        out_shape=jax.ShapeDtypeStruct((nb, 1, B_TILE), jnp.float32),
        scratch_shapes=[
            pltpu.VMEM((N_LABELS, N_INPUT), jnp.bfloat16),
            pltpu.VMEM((N_LABELS, N_INPUT), jnp.bfloat16),
            pltpu.VMEM((N_LABELS, N_INPUT), jnp.bfloat16),
            pltpu.VMEM((1, N_INPUT), jnp.float32),
            pltpu.SMEM((1, 1), jnp.float32),
        ],
        interpret=interpret,
    )(x, y2, pxr2, wt)
    return loss.reshape(BATCH)


def kernel(x, y, ind_x, px_r, W):
    loss = _run(x, y, px_r, W)
    zero = jnp.asarray(0.0, dtype=jnp.float32)
    return (loss, zero, zero)


# speedup vs baseline: 5.0001x; 1.0577x over previous
import functools

import jax
import jax.numpy as jnp
from jax.experimental import pallas as pl
from jax.experimental.pallas import tpu as pltpu

N_INPUT = 20000
N_LABELS = 64
BATCH = 4096
EPS = 1e-8
B_TILE = 32

_LG_N = (
    655.1778003977308,
    651.7861284548891,
    243.1516405664637,
    40.31491809436625,
    2.5066282746310007,
)
_LG_D = (6.0, 11.0, 6.0, 1.0)


def _lgamma_pos(a):
    n = jnp.float32(_LG_N[-1])
    for c in _LG_N[-2::-1]:
        n = n * a + jnp.float32(c)
    d = jnp.float32(_LG_D[-1])
    for c in _LG_D[-2::-1]:
        d = d * a + jnp.float32(c)
    d = d * a
    t = a + 4.0
    return (a - 0.5) * jnp.log(t) - t + jnp.log(n / d)


_SP3_P = (10.449703348243359, 10.238049794415314, 2.5066282746310007)
_SP3_PS = (23.19438141728967, 15.251306343677316, 2.5066282746310007)


def _loss_kernel(
    x_ref, y_ref, pxr_ref, wt_ref, out_ref, hi_ref, md_ref, lo_ref, th_ref, c_ref
):
    i = pl.program_id(0)

    @pl.when(i == 0)
    def _init():
        sp = jax.nn.softplus(wt_ref[...])
        hi = sp.astype(jnp.bfloat16)
        r1 = sp - hi.astype(jnp.float32)
        md = r1.astype(jnp.bfloat16)
        lo = (r1 - md.astype(jnp.float32)).astype(jnp.bfloat16)
        hi_ref[...] = hi
        md_ref[...] = md
        lo_ref[...] = lo
        theta = jnp.exp(pxr_ref[...])
        th_ref[...] = theta
        c_ref[0, 0] = jnp.sum(
            theta * jnp.log(theta + EPS) - _lgamma_pos(theta) - theta + 1.0
        )

    xb = x_ref[...]
    yb = y_ref[0, 0, :]
    labels = jax.lax.broadcasted_iota(jnp.int32, (B_TILE, N_LABELS), 1)
    onehot = (yb[:, None] == labels).astype(jnp.bfloat16)

    def _sel(ref):
        return jax.lax.dot_general(
            onehot,
            ref[...],
            dimension_numbers=(((1,), (0,)), ((), ())),
            preferred_element_type=jnp.float32,
        )

    px_scale = (_sel(hi_ref) + _sel(md_ref)) + _sel(lo_ref)

    th = th_ref[...]
    te = th + EPS
    th2 = th + 2.0

    lib = jnp.sum(xb, axis=1, keepdims=True)
    mu = lib * px_scale
    a1 = xb + th
    t1 = xb + th2
    t2 = xb + 3.0
    p1 = (_SP3_P[2] * a1 + _SP3_P[1]) * a1 + _SP3_P[0]
    q1 = a1 * (a1 + 1.0)
    p2 = (_SP3_PS[2] * xb + _SP3_PS[1]) * xb + _SP3_PS[0]
    v = xb + 1.5
    q2 = v * v - 0.25
    lga = jnp.log(t1 / (te + mu))
    lgb = jnp.log((mu + EPS) / t2)
    lgc = jnp.log((p1 * q2) / (q1 * p2) * jax.lax.rsqrt(t1 * t2))
    contrib = a1 * lga + xb * lgb + lgc
    out_ref[0, 0, :] = -(jnp.sum(contrib, axis=1) + c_ref[0, 0])


@functools.partial(jax.jit, static_argnames=("interpret",))
def _run(x, y, px_r, W, interpret=False):
    nb = BATCH // B_TILE
    y2 = y.reshape(nb, 1, B_TILE)
    pxr2 = px_r.reshape(1, N_INPUT)
    wt = W.T

    loss = pl.pallas_call(
        _loss_kernel,
        grid=(nb,),
        in_specs=[
            pl.BlockSpec((B_TILE, N_INPUT), lambda i: (i, 0)),
            pl.BlockSpec((1, 1, B_TILE), lambda i: (i, 0, 0)),
            pl.BlockSpec((1, N_INPUT), lambda i: (0, 0)),
            pl.BlockSpec((N_LABELS, N_INPUT), lambda i: (0, 0)),
        ],
        out_specs=pl.BlockSpec((1, 1, B_TILE), lambda i: (i, 0, 0)),
        out_shape=jax.ShapeDtypeStruct((nb, 1, B_TILE), jnp.float32),
        scratch_shapes=[
            pltpu.VMEM((N_LABELS, N_INPUT), jnp.bfloat16),
            pltpu.VMEM((N_LABELS, N_INPUT), jnp.bfloat16),
            pltpu.VMEM((N_LABELS, N_INPUT), jnp.bfloat16),
            pltpu.VMEM((1, N_INPUT), jnp.float32),
            pltpu.SMEM((1, 1), jnp.float32),
        ],
        interpret=interpret,
    )(x, y2, pxr2, wt)
    return loss.reshape(BATCH)


def kernel(x, y, ind_x, px_r, W):
    loss = _run(x, y, px_r, W)
    zero = jnp.asarray(0.0, dtype=jnp.float32)
    return (loss, zero, zero)


# speedup vs baseline: 5.0898x; 1.0179x over previous
import functools

import jax
import jax.numpy as jnp
from jax.experimental import pallas as pl
from jax.experimental.pallas import tpu as pltpu

N_INPUT = 20000
N_LABELS = 64
BATCH = 4096
EPS = 1e-8
B_TILE = 64

_LG_N = (
    655.1778003977308,
    651.7861284548891,
    243.1516405664637,
    40.31491809436625,
    2.5066282746310007,
)
_LG_D = (6.0, 11.0, 6.0, 1.0)


def _lgamma_pos(a):
    n = jnp.float32(_LG_N[-1])
    for c in _LG_N[-2::-1]:
        n = n * a + jnp.float32(c)
    d = jnp.float32(_LG_D[-1])
    for c in _LG_D[-2::-1]:
        d = d * a + jnp.float32(c)
    d = d * a
    t = a + 4.0
    return (a - 0.5) * jnp.log(t) - t + jnp.log(n / d)


_SP3_P = (10.449703348243359, 10.238049794415314, 2.5066282746310007)
_SP3_PS = (23.19438141728967, 15.251306343677316, 2.5066282746310007)


def _loss_kernel(
    x_ref, y_ref, pxr_ref, wt_ref, out_ref, hi_ref, md_ref, lo_ref, th_ref, c_ref
):
    i = pl.program_id(0)

    @pl.when(i == 0)
    def _init():
        sp = jax.nn.softplus(wt_ref[...])
        hi = sp.astype(jnp.bfloat16)
        r1 = sp - hi.astype(jnp.float32)
        md = r1.astype(jnp.bfloat16)
        lo = (r1 - md.astype(jnp.float32)).astype(jnp.bfloat16)
        hi_ref[...] = hi
        md_ref[...] = md
        lo_ref[...] = lo
        theta = jnp.exp(pxr_ref[...])
        th_ref[...] = theta
        c_ref[0, 0] = jnp.sum(
            theta * jnp.log(theta + EPS) - _lgamma_pos(theta) - theta + 1.0
        )

    xb = x_ref[...]
    yb = y_ref[0, 0, :]
    labels = jax.lax.broadcasted_iota(jnp.int32, (B_TILE, N_LABELS), 1)
    onehot = (yb[:, None] == labels).astype(jnp.bfloat16)

    def _sel(ref):
        return jax.lax.dot_general(
            onehot,
            ref[...],
            dimension_numbers=(((1,), (0,)), ((), ())),
            preferred_element_type=jnp.float32,
        )

    px_scale = (_sel(hi_ref) + _sel(md_ref)) + _sel(lo_ref)

    th = th_ref[...]
    te = th + EPS
    th2 = th + 2.0

    lib = jnp.sum(xb, axis=1, keepdims=True)
    mu = lib * px_scale
    a1 = xb + th
    t1 = xb + th2
    t2 = xb + 3.0
    p1 = (_SP3_P[2] * a1 + _SP3_P[1]) * a1 + _SP3_P[0]
    q1 = a1 * (a1 + 1.0)
    p2 = (_SP3_PS[2] * xb + _SP3_PS[1]) * xb + _SP3_PS[0]
    v = xb + 1.5
    q2 = v * v - 0.25
    lga = jnp.log(t1 / (te + mu))
    lgb = jnp.log((mu + EPS) / t2)
    lgc = jnp.log((p1 * q2) / (q1 * p2) * jax.lax.rsqrt(t1 * t2))
    contrib = a1 * lga + xb * lgb + lgc
    out_ref[0, 0, :] = -(jnp.sum(contrib, axis=1) + c_ref[0, 0])


@functools.partial(jax.jit, static_argnames=("interpret",))
def _run(x, y, px_r, W, interpret=False):
    nb = BATCH // B_TILE
    y2 = y.reshape(nb, 1, B_TILE)
    pxr2 = px_r.reshape(1, N_INPUT)
    wt = W.T

    loss = pl.pallas_call(
        _loss_kernel,
        grid=(nb,),
        in_specs=[
            pl.BlockSpec((B_TILE, N_INPUT), lambda i: (i, 0)),
            pl.BlockSpec((1, 1, B_TILE), lambda i: (i, 0, 0)),
            pl.BlockSpec((1, N_INPUT), lambda i: (0, 0)),
            pl.BlockSpec((N_LABELS, N_INPUT), lambda i: (0, 0)),
        ],
        out_specs=pl.BlockSpec((1, 1, B_TILE), lambda i: (i, 0, 0)),
        out_shape=jax.ShapeDtypeStruct((nb, 1, B_TILE), jnp.float32),
        scratch_shapes=[
            pltpu.VMEM((N_LABELS, N_INPUT), jnp.bfloat16),
            pltpu.VMEM((N_LABELS, N_INPUT), jnp.bfloat16),
            pltpu.VMEM((N_LABELS, N_INPUT), jnp.bfloat16),
            pltpu.VMEM((1, N_INPUT), jnp.float32),
            pltpu.SMEM((1, 1), jnp.float32),
        ],
        interpret=interpret,
    )(x, y2, pxr2, wt)
    return loss.reshape(BATCH)


def kernel(x, y, ind_x, px_r, W):
    loss = _run(x, y, px_r, W)
    zero = jnp.asarray(0.0, dtype=jnp.float32)
    return (loss, zero, zero)


# speedup vs baseline: 5.1203x; 1.0060x over previous
import functools

import jax
import jax.numpy as jnp
from jax.experimental import pallas as pl
from jax.experimental.pallas import tpu as pltpu

N_INPUT = 20000
N_LABELS = 64
BATCH = 4096
EPS = 1e-8
B_TILE = 64

_LG_N = (
    655.1778003977308,
    651.7861284548891,
    243.1516405664637,
    40.31491809436625,
    2.5066282746310007,
)
_LG_D = (6.0, 11.0, 6.0, 1.0)


def _lgamma_pos(a):
    n = jnp.float32(_LG_N[-1])
    for c in _LG_N[-2::-1]:
        n = n * a + jnp.float32(c)
    d = jnp.float32(_LG_D[-1])
    for c in _LG_D[-2::-1]:
        d = d * a + jnp.float32(c)
    d = d * a
    t = a + 4.0
    return (a - 0.5) * jnp.log(t) - t + jnp.log(n / d)


_SP3_P = (10.449703348243359, 10.238049794415314, 2.5066282746310007)
_SP3_PS = (23.19438141728967, 15.251306343677316, 2.5066282746310007)


def _loss_kernel(x_ref, y_ref, pxr_ref, wt_ref, out_ref, tab_ref, th_ref, c_ref):
    i = pl.program_id(0)

    @pl.when(i == 0)
    def _init():
        sp = jax.nn.softplus(wt_ref[...])
        hi = sp.astype(jnp.bfloat16)
        r1 = sp - hi.astype(jnp.float32)
        md = r1.astype(jnp.bfloat16)
        lo = (r1 - md.astype(jnp.float32)).astype(jnp.bfloat16)
        tab_ref[0:N_LABELS, :] = hi
        tab_ref[N_LABELS : 2 * N_LABELS, :] = md
        tab_ref[2 * N_LABELS :, :] = lo
        theta = jnp.exp(pxr_ref[...])
        th_ref[...] = theta
        c_ref[0, 0] = jnp.sum(
            theta * jnp.log(theta + EPS) - _lgamma_pos(theta) - theta + 1.0
        )

    xb = x_ref[...]
    yb = y_ref[0, 0, :]
    labels = jax.lax.broadcasted_iota(jnp.int32, (B_TILE, 3 * N_LABELS), 1)
    onehot = (yb[:, None] == labels % N_LABELS).astype(jnp.bfloat16)
    px_scale = jax.lax.dot_general(
        onehot,
        tab_ref[...],
        dimension_numbers=(((1,), (0,)), ((), ())),
        preferred_element_type=jnp.float32,
    )

    th = th_ref[...]
    te = th + EPS
    th2 = th + 2.0

    lib = jnp.sum(xb, axis=1, keepdims=True)
    mu = lib * px_scale
    a1 = xb + th
    t1 = xb + th2
    t2 = xb + 3.0
    p1 = (_SP3_P[2] * a1 + _SP3_P[1]) * a1 + _SP3_P[0]
    q1 = a1 * (a1 + 1.0)
    p2 = (_SP3_PS[2] * xb + _SP3_PS[1]) * xb + _SP3_PS[0]
    v = xb + 1.5
    q2 = v * v - 0.25
    tm = te + mu
    w = q1 * p2
    z1 = tm * t2
    r = 1.0 / (z1 * w)
    u = t2 * w
    lga = jnp.log(t1 * u * r)
    lgb = jnp.log((mu + EPS) * (tm * w) * r)
    lgc = jnp.log((p1 * q2) * z1 * r * jax.lax.rsqrt(t1 * t2))
    contrib = a1 * lga + xb * lgb + lgc
    out_ref[0, 0, :] = -(jnp.sum(contrib, axis=1) + c_ref[0, 0])


@functools.partial(jax.jit, static_argnames=("interpret",))
def _run(x, y, px_r, W, interpret=False):
    nb = BATCH // B_TILE
    y2 = y.reshape(nb, 1, B_TILE)
    pxr2 = px_r.reshape(1, N_INPUT)
    wt = W.T

    loss = pl.pallas_call(
        _loss_kernel,
        grid=(nb,),
        in_specs=[
            pl.BlockSpec((B_TILE, N_INPUT), lambda i: (i, 0)),
            pl.BlockSpec((1, 1, B_TILE), lambda i: (i, 0, 0)),
            pl.BlockSpec((1, N_INPUT), lambda i: (0, 0)),
            pl.BlockSpec((N_LABELS, N_INPUT), lambda i: (0, 0)),
        ],
        out_specs=pl.BlockSpec((1, 1, B_TILE), lambda i: (i, 0, 0)),
        out_shape=jax.ShapeDtypeStruct((nb, 1, B_TILE), jnp.float32),
        scratch_shapes=[
            pltpu.VMEM((3 * N_LABELS, N_INPUT), jnp.bfloat16),
            pltpu.VMEM((1, N_INPUT), jnp.float32),
            pltpu.SMEM((1, 1), jnp.float32),
        ],
        interpret=interpret,
    )(x, y2, pxr2, wt)
    return loss.reshape(BATCH)


def kernel(x, y, ind_x, px_r, W):
    loss = _run(x, y, px_r, W)
    zero = jnp.asarray(0.0, dtype=jnp.float32)
    return (loss, zero, zero)
